# Initial kernel scaffold; baseline (speedup 1.0000x reference)
import jax
import jax.numpy as jnp
from jax.experimental import pallas as pl


def kernel(x, conn, weight, bias_param):
    raise NotImplementedError("write your pallas kernel here")


# speedup vs baseline: 6.0971x; 6.0971x over previous
import functools

import jax
import jax.numpy as jnp
from jax import lax
from jax.experimental import pallas as pl
from jax.experimental.pallas import tpu as pltpu
from jax.experimental.pallas import tpu_sc as plsc

IN_F = 2048
OUT_F = 2048
NCONN = 16

NUM_CORES = 2
NUM_SUBCORES = 16
NW = NUM_CORES * NUM_SUBCORES
ROWS_PER_W = OUT_F // NW
SUB = 16
NCHUNK = ROWS_PER_W // SUB


def _build_mt(conn_i32, w_flat, zeros_blk):
    mesh = plsc.VectorSubcoreMesh(core_axis_name="c", subcore_axis_name="s")

    @functools.partial(
        pl.kernel,
        mesh=mesh,
        out_type=jax.ShapeDtypeStruct((OUT_F, IN_F), jnp.float32),
        scratch_types=[
            pltpu.VMEM((SUB, IN_F), jnp.float32),
            pltpu.VMEM((SUB, IN_F), jnp.float32),
            pltpu.VMEM((ROWS_PER_W * NCONN,), jnp.int32),
            pltpu.VMEM((ROWS_PER_W * NCONN,), jnp.float32),
            pltpu.SemaphoreType.DMA,
            pltpu.SemaphoreType.DMA,
        ],
        compiler_params=pltpu.CompilerParams(needs_layout_passes=False),
    )
    def k(conn_hbm, w_hbm, z_hbm, mt_hbm, blk0, blk1, idx_v, wv, sem0, sem1):
        wid = lax.axis_index("s") * NUM_CORES + lax.axis_index("c")
        base = wid * ROWS_PER_W
        blks = (blk0, blk1)
        sems = (sem0, sem1)
        pltpu.sync_copy(conn_hbm.at[pl.ds(base * NCONN, ROWS_PER_W * NCONN)],
                        idx_v)
        pltpu.sync_copy(w_hbm.at[pl.ds(base * NCONN, ROWS_PER_W * NCONN)], wv)
        pltpu.sync_copy(z_hbm, blk0)
        pltpu.sync_copy(z_hbm, blk1)
        zvec = jnp.zeros((NCONN,), jnp.float32)
        copies = [None, None]
        for c in range(NCHUNK):
            b = c % 2
            blk = blks[b]
            if copies[b] is not None:
                copies[b].wait()
                for r in range(SUB):
                    o = ((c - 2) * SUB + r) * NCONN
                    rvec = jnp.full((NCONN,), r, jnp.int32)
                    plsc.store_scatter(blk, [rvec, idx_v[pl.ds(o, NCONN)]],
                                       zvec)
            for r in range(SUB):
                o = (c * SUB + r) * NCONN
                idx = idx_v[pl.ds(o, NCONN)]
                w = wv[pl.ds(o, NCONN)]
                rvec = jnp.full((NCONN,), r, jnp.int32)
                plsc.addupdate_scatter(blk, [rvec, idx], w)
            copies[b] = pltpu.async_copy(
                blk, mt_hbm.at[pl.ds(base + c * SUB, SUB)], sems[b])
        copies[0].wait()
        copies[1].wait()

    return k(conn_i32, w_flat, zeros_blk)


def _matmul(x2d, mt, bias2d):
    A_BLK = 512

    def body(x_ref, mt_ref, b_ref, o_ref):
        acc = lax.dot_general(
            x_ref[...], mt_ref[...],
            (((1,), (1,)), ((), ())),
            preferred_element_type=jnp.float32,
        )
        o_ref[...] = acc + b_ref[...]

    return pl.pallas_call(
        body,
        grid=(OUT_F // A_BLK,),
        in_specs=[
            pl.BlockSpec((2048, IN_F), lambda i: (0, 0)),
            pl.BlockSpec((A_BLK, IN_F), lambda i: (i, 0)),
            pl.BlockSpec((1, A_BLK), lambda i: (0, i)),
        ],
        out_specs=pl.BlockSpec((2048, A_BLK), lambda i: (0, i)),
        out_shape=jax.ShapeDtypeStruct((2048, OUT_F), jnp.float32),
    )(x2d, mt, bias2d)


def kernel(x, conn, weight, bias_param):
    conn_i = conn.astype(jnp.int32)
    w_flat = weight.astype(jnp.float32).reshape(-1)
    zeros_blk = jnp.zeros((SUB, IN_F), jnp.float32)
    mt = _build_mt(conn_i, w_flat, zeros_blk)
    out = _matmul(x[0], mt, bias_param.reshape(1, OUT_F))
    return out[None]


# speedup vs baseline: 6.2092x; 1.0184x over previous
import functools

import jax
import jax.numpy as jnp
from jax import lax
from jax.experimental import pallas as pl
from jax.experimental.pallas import tpu as pltpu
from jax.experimental.pallas import tpu_sc as plsc

IN_F = 2048
OUT_F = 2048
NCONN = 16

NUM_CORES = 2
NUM_SUBCORES = 16
NW = NUM_CORES * NUM_SUBCORES
ROWS_PER_W = OUT_F // NW
SUB = 16
NCHUNK = ROWS_PER_W // SUB


def _build_mt(conn_i32, weight, zeros_blk):
    mesh = plsc.VectorSubcoreMesh(core_axis_name="c", subcore_axis_name="s")

    @functools.partial(
        pl.kernel,
        mesh=mesh,
        out_type=jax.ShapeDtypeStruct((OUT_F, IN_F), jnp.float32),
        scratch_types=[
            pltpu.VMEM((SUB, IN_F), jnp.float32),
            pltpu.VMEM((SUB, IN_F), jnp.float32),
            pltpu.VMEM((ROWS_PER_W * NCONN,), jnp.int32),
            pltpu.VMEM((ROWS_PER_W, NCONN), jnp.float32),
            pltpu.SemaphoreType.DMA,
            pltpu.SemaphoreType.DMA,
            pltpu.SemaphoreType.DMA,
            pltpu.SemaphoreType.DMA,
        ],
        compiler_params=pltpu.CompilerParams(needs_layout_passes=False),
    )
    def k(conn_hbm, w_hbm, z_hbm, mt_hbm, blk0, blk1, idx_v, wv, s0, s1, s2,
          s3):
        wid = lax.axis_index("s") * NUM_CORES + lax.axis_index("c")
        base = wid * ROWS_PER_W
        blks = (blk0, blk1)
        sems = (s0, s1)
        z0 = pltpu.async_copy(z_hbm, blk0, s2)
        z1 = pltpu.async_copy(z_hbm, blk1, s3)
        pltpu.sync_copy(conn_hbm.at[pl.ds(base * NCONN, ROWS_PER_W * NCONN)],
                        idx_v)
        pltpu.sync_copy(w_hbm.at[pl.ds(base, ROWS_PER_W)], wv)
        z0.wait()
        z1.wait()
        zvec = jnp.zeros((NCONN,), jnp.float32)
        copies = [None, None]
        for c in range(NCHUNK):
            b = c % 2
            blk = blks[b]
            if copies[b] is not None:
                copies[b].wait()
                for r in range(SUB):
                    o = ((c - 2) * SUB + r) * NCONN
                    rvec = jnp.full((NCONN,), r, jnp.int32)
                    plsc.store_scatter(blk, [rvec, idx_v[pl.ds(o, NCONN)]],
                                       zvec)
            for r in range(SUB):
                o = (c * SUB + r) * NCONN
                idx = idx_v[pl.ds(o, NCONN)]
                w = wv[c * SUB + r]
                rvec = jnp.full((NCONN,), r, jnp.int32)
                plsc.addupdate_scatter(blk, [rvec, idx], w)
            copies[b] = pltpu.async_copy(
                blk, mt_hbm.at[pl.ds(base + c * SUB, SUB)], sems[b])
        copies[0].wait()
        copies[1].wait()

    return k(conn_i32, weight, zeros_blk)


def _matmul(x2d, mt, bias):
    A_BLK = 512

    def body(x_ref, mt_ref, b_ref, o_ref):
        acc = lax.dot_general(
            x_ref[...], mt_ref[...],
            (((1,), (1,)), ((), ())),
            preferred_element_type=jnp.float32,
        )
        o_ref[...] = acc + b_ref[...]

    return pl.pallas_call(
        body,
        grid=(OUT_F // A_BLK,),
        in_specs=[
            pl.BlockSpec((2048, IN_F), lambda i: (0, 0)),
            pl.BlockSpec((A_BLK, IN_F), lambda i: (i, 0)),
            pl.BlockSpec((A_BLK,), lambda i: (i,)),
        ],
        out_specs=pl.BlockSpec((2048, A_BLK), lambda i: (0, i)),
        out_shape=jax.ShapeDtypeStruct((2048, OUT_F), jnp.float32),
    )(x2d, mt, bias)


def kernel(x, conn, weight, bias_param):
    conn_i = conn.astype(jnp.int32)
    w32 = weight.astype(jnp.float32)
    zeros_blk = jnp.zeros((SUB, IN_F), jnp.float32)
    mt = _build_mt(conn_i, w32, zeros_blk)
    out = _matmul(x[0], mt, bias_param)
    return out[None]


# speedup vs baseline: 6.3312x; 1.0196x over previous
import functools

import jax
import jax.numpy as jnp
from jax import lax
from jax.experimental import pallas as pl
from jax.experimental.pallas import tpu as pltpu
from jax.experimental.pallas import tpu_sc as plsc

IN_F = 2048
OUT_F = 2048
NCONN = 16

NUM_CORES = 2
NUM_SUBCORES = 16
NW = NUM_CORES * NUM_SUBCORES
ROWS_PER_W = OUT_F // NW
SUB = 16
NCHUNK = ROWS_PER_W // SUB


def _build_mt(conn_i32, weight, zeros_blk):
    mesh = plsc.VectorSubcoreMesh(core_axis_name="c", subcore_axis_name="s")

    @functools.partial(
        pl.kernel,
        mesh=mesh,
        out_type=jax.ShapeDtypeStruct((OUT_F, IN_F), jnp.float32),
        scratch_types=[
            pltpu.VMEM((SUB, IN_F), jnp.float32),
            pltpu.VMEM((SUB, IN_F), jnp.float32),
            pltpu.VMEM((ROWS_PER_W * NCONN,), jnp.int32),
            pltpu.VMEM((ROWS_PER_W, NCONN), jnp.float32),
            pltpu.SemaphoreType.DMA,
            pltpu.SemaphoreType.DMA,
            pltpu.SemaphoreType.DMA,
            pltpu.SemaphoreType.DMA,
        ],
        compiler_params=pltpu.CompilerParams(needs_layout_passes=False),
    )
    def k(conn_hbm, w_hbm, z_hbm, mt_hbm, blk0, blk1, idx_v, wv, s0, s1, s2,
          s3):
        wid = lax.axis_index("s") * NUM_CORES + lax.axis_index("c")
        base = wid * ROWS_PER_W
        blks = (blk0, blk1)
        sems = (s0, s1)
        z0 = pltpu.async_copy(z_hbm, blk0, s2)
        z1 = pltpu.async_copy(z_hbm, blk1, s3)
        pltpu.sync_copy(conn_hbm.at[pl.ds(base * NCONN, ROWS_PER_W * NCONN)],
                        idx_v)
        pltpu.sync_copy(w_hbm.at[pl.ds(base, ROWS_PER_W)], wv)
        z0.wait()
        z1.wait()
        zvec = jnp.zeros((NCONN,), jnp.float32)
        copies = [None, None]
        for c in range(NCHUNK):
            b = c % 2
            blk = blks[b]
            if copies[b] is not None:
                copies[b].wait()

                def _restore(r, _, blk=blk, c=c):
                    o = ((c - 2) * SUB + r) * NCONN
                    rvec = jnp.full((NCONN,), r, jnp.int32)
                    plsc.store_scatter(blk, [rvec, idx_v[pl.ds(o, NCONN)]],
                                       zvec)
                    return _

                lax.fori_loop(0, SUB, _restore, None, unroll=4)

            def _scatter(r, _, blk=blk, c=c):
                o = (c * SUB + r) * NCONN
                idx = idx_v[pl.ds(o, NCONN)]
                w = wv[c * SUB + r]
                rvec = jnp.full((NCONN,), r, jnp.int32)
                plsc.addupdate_scatter(blk, [rvec, idx], w)
                return _

            lax.fori_loop(0, SUB, _scatter, None, unroll=4)
            copies[b] = pltpu.async_copy(
                blk, mt_hbm.at[pl.ds(base + c * SUB, SUB)], sems[b])
        copies[0].wait()
        copies[1].wait()

    return k(conn_i32, weight, zeros_blk)


def _matmul(x2d, mt, bias):
    A_BLK = 256

    def body(x_ref, mt_ref, b_ref, o_ref):
        acc = lax.dot_general(
            x_ref[...], mt_ref[...],
            (((1,), (1,)), ((), ())),
            preferred_element_type=jnp.float32,
        )
        o_ref[...] = acc + b_ref[...]

    return pl.pallas_call(
        body,
        grid=(OUT_F // A_BLK,),
        in_specs=[
            pl.BlockSpec((2048, IN_F), lambda i: (0, 0)),
            pl.BlockSpec((A_BLK, IN_F), lambda i: (i, 0)),
            pl.BlockSpec((A_BLK,), lambda i: (i,)),
        ],
        out_specs=pl.BlockSpec((2048, A_BLK), lambda i: (0, i)),
        out_shape=jax.ShapeDtypeStruct((2048, OUT_F), jnp.float32),
    )(x2d, mt, bias)


def kernel(x, conn, weight, bias_param):
    conn_i = conn.astype(jnp.int32)
    w32 = weight.astype(jnp.float32)
    zeros_blk = jnp.zeros((SUB, IN_F), jnp.float32)
    mt = _build_mt(conn_i, w32, zeros_blk)
    out = _matmul(x[0], mt, bias_param)
    return out[None]


# speedup vs baseline: 6.5953x; 1.0417x over previous
import functools

import jax
import jax.numpy as jnp
from jax import lax
from jax.experimental import pallas as pl
from jax.experimental.pallas import tpu as pltpu
from jax.experimental.pallas import tpu_sc as plsc

IN_F = 2048
OUT_F = 2048
NCONN = 16

NUM_CORES = 2
NUM_SUBCORES = 16
NW = NUM_CORES * NUM_SUBCORES
ROWS_PER_W = OUT_F // NW
SUB = 8
NCHUNK = ROWS_PER_W // SUB


def _build_mt(conn_i32, weight, zeros_blk):
    mesh = plsc.VectorSubcoreMesh(core_axis_name="c", subcore_axis_name="s")

    @functools.partial(
        pl.kernel,
        mesh=mesh,
        out_type=jax.ShapeDtypeStruct((OUT_F, IN_F), jnp.float32),
        scratch_types=[
            pltpu.VMEM((SUB, IN_F), jnp.float32),
            pltpu.VMEM((SUB, IN_F), jnp.float32),
            pltpu.VMEM((ROWS_PER_W * NCONN,), jnp.int32),
            pltpu.VMEM((ROWS_PER_W, NCONN), jnp.float32),
            pltpu.SemaphoreType.DMA,
            pltpu.SemaphoreType.DMA,
            pltpu.SemaphoreType.DMA,
            pltpu.SemaphoreType.DMA,
        ],
        compiler_params=pltpu.CompilerParams(needs_layout_passes=False),
    )
    def k(conn_hbm, w_hbm, z_hbm, mt_hbm, blk0, blk1, idx_v, wv, s0, s1, s2,
          s3):
        wid = lax.axis_index("s") * NUM_CORES + lax.axis_index("c")
        base = wid * ROWS_PER_W
        blks = (blk0, blk1)
        sems = (s0, s1)
        z0 = pltpu.async_copy(z_hbm, blk0, s2)
        z1 = pltpu.async_copy(z_hbm, blk1, s3)
        pltpu.sync_copy(conn_hbm.at[pl.ds(base * NCONN, ROWS_PER_W * NCONN)],
                        idx_v)
        pltpu.sync_copy(w_hbm.at[pl.ds(base, ROWS_PER_W)], wv)
        z0.wait()
        z1.wait()
        zvec = jnp.zeros((NCONN,), jnp.float32)
        copies = [None, None]
        for c in range(NCHUNK):
            b = c % 2
            blk = blks[b]
            if copies[b] is not None:
                copies[b].wait()

                def _restore(r, _, blk=blk, c=c):
                    o = ((c - 2) * SUB + r) * NCONN
                    rvec = jnp.full((NCONN,), r, jnp.int32)
                    plsc.store_scatter(blk, [rvec, idx_v[pl.ds(o, NCONN)]],
                                       zvec)
                    return _

                lax.fori_loop(0, SUB, _restore, None, unroll=4)

            def _scatter(r, _, blk=blk, c=c):
                o = (c * SUB + r) * NCONN
                idx = idx_v[pl.ds(o, NCONN)]
                w = wv[c * SUB + r]
                rvec = jnp.full((NCONN,), r, jnp.int32)
                plsc.addupdate_scatter(blk, [rvec, idx], w)
                return _

            lax.fori_loop(0, SUB, _scatter, None, unroll=4)
            copies[b] = pltpu.async_copy(
                blk, mt_hbm.at[pl.ds(base + c * SUB, SUB)], sems[b])
        copies[0].wait()
        copies[1].wait()

    return k(conn_i32, weight, zeros_blk)


def _matmul(x2d, mt, bias):
    A_BLK = 256

    def body(x_ref, mt_ref, b_ref, o_ref):
        acc = lax.dot_general(
            x_ref[...], mt_ref[...],
            (((1,), (1,)), ((), ())),
            preferred_element_type=jnp.float32,
        )
        o_ref[...] = acc + b_ref[...]

    return pl.pallas_call(
        body,
        grid=(OUT_F // A_BLK,),
        in_specs=[
            pl.BlockSpec((2048, IN_F), lambda i: (0, 0)),
            pl.BlockSpec((A_BLK, IN_F), lambda i: (i, 0)),
            pl.BlockSpec((A_BLK,), lambda i: (i,)),
        ],
        out_specs=pl.BlockSpec((2048, A_BLK), lambda i: (0, i)),
        out_shape=jax.ShapeDtypeStruct((2048, OUT_F), jnp.float32),
    )(x2d, mt, bias)


def kernel(x, conn, weight, bias_param):
    conn_i = conn.astype(jnp.int32)
    zeros_blk = jnp.zeros((SUB, IN_F), jnp.float32)
    mt = _build_mt(conn_i, weight, zeros_blk)
    out = _matmul(x[0], mt, bias_param)
    return out[None]


# speedup vs baseline: 7.1201x; 1.0796x over previous
import functools

import jax
import jax.numpy as jnp
from jax import lax
from jax.experimental import pallas as pl
from jax.experimental.pallas import tpu as pltpu
from jax.experimental.pallas import tpu_sc as plsc

IN_F = 2048
OUT_F = 2048
NCONN = 16

NUM_CORES = 2
NUM_SUBCORES = 16
NW = NUM_CORES * NUM_SUBCORES
ROWS_PER_W = OUT_F // NW
SUB = 4
NCHUNK = ROWS_PER_W // SUB


def _build_mt(conn_i32, weight, zeros_blk):
    mesh = plsc.VectorSubcoreMesh(core_axis_name="c", subcore_axis_name="s")

    @functools.partial(
        pl.kernel,
        mesh=mesh,
        out_type=jax.ShapeDtypeStruct((OUT_F, IN_F), jnp.float32),
        scratch_types=[
            pltpu.VMEM((SUB, IN_F), jnp.float32),
            pltpu.VMEM((SUB, IN_F), jnp.float32),
            pltpu.VMEM((ROWS_PER_W * NCONN,), jnp.int32),
            pltpu.VMEM((ROWS_PER_W, NCONN), jnp.float32),
            pltpu.SemaphoreType.DMA,
            pltpu.SemaphoreType.DMA,
            pltpu.SemaphoreType.DMA,
            pltpu.SemaphoreType.DMA,
        ],
        compiler_params=pltpu.CompilerParams(needs_layout_passes=False),
    )
    def k(conn_hbm, w_hbm, z_hbm, mt_hbm, blk0, blk1, idx_v, wv, s0, s1, s2,
          s3):
        wid = lax.axis_index("s") * NUM_CORES + lax.axis_index("c")
        base = wid * ROWS_PER_W
        blks = (blk0, blk1)
        sems = (s0, s1)
        z0 = pltpu.async_copy(z_hbm, blk0, s2)
        z1 = pltpu.async_copy(z_hbm, blk1, s3)
        pltpu.sync_copy(conn_hbm.at[pl.ds(base * NCONN, ROWS_PER_W * NCONN)],
                        idx_v)
        pltpu.sync_copy(w_hbm.at[pl.ds(base, ROWS_PER_W)], wv)
        z0.wait()
        z1.wait()
        zvec = jnp.zeros((NCONN,), jnp.float32)
        copies = [None, None]
        for c in range(NCHUNK):
            b = c % 2
            blk = blks[b]
            if copies[b] is not None:
                copies[b].wait()

                def _restore(r, _, blk=blk, c=c):
                    o = ((c - 2) * SUB + r) * NCONN
                    rvec = jnp.full((NCONN,), r, jnp.int32)
                    plsc.store_scatter(blk, [rvec, idx_v[pl.ds(o, NCONN)]],
                                       zvec)
                    return _

                lax.fori_loop(0, SUB, _restore, None, unroll=4)

            def _scatter(r, _, blk=blk, c=c):
                o = (c * SUB + r) * NCONN
                idx = idx_v[pl.ds(o, NCONN)]
                w = wv[c * SUB + r]
                rvec = jnp.full((NCONN,), r, jnp.int32)
                plsc.addupdate_scatter(blk, [rvec, idx], w)
                return _

            lax.fori_loop(0, SUB, _scatter, None, unroll=4)
            copies[b] = pltpu.async_copy(
                blk, mt_hbm.at[pl.ds(base + c * SUB, SUB)], sems[b])
        copies[0].wait()
        copies[1].wait()

    return k(conn_i32, weight, zeros_blk)


def _matmul(x2d, mt, bias):
    A_BLK = 256

    def body(x_ref, mt_ref, b_ref, o_ref):
        acc = lax.dot_general(
            x_ref[...], mt_ref[...],
            (((1,), (1,)), ((), ())),
            preferred_element_type=jnp.float32,
        )
        o_ref[...] = acc + b_ref[...]

    return pl.pallas_call(
        body,
        grid=(OUT_F // A_BLK,),
        in_specs=[
            pl.BlockSpec((2048, IN_F), lambda i: (0, 0)),
            pl.BlockSpec((A_BLK, IN_F), lambda i: (i, 0)),
            pl.BlockSpec((A_BLK,), lambda i: (i,)),
        ],
        out_specs=pl.BlockSpec((2048, A_BLK), lambda i: (0, i)),
        out_shape=jax.ShapeDtypeStruct((2048, OUT_F), jnp.float32),
    )(x2d, mt, bias)


def kernel(x, conn, weight, bias_param):
    conn_i = conn.astype(jnp.int32)
    zeros_blk = jnp.zeros((SUB, IN_F), jnp.float32)
    mt = _build_mt(conn_i, weight, zeros_blk)
    out = _matmul(x[0], mt, bias_param)
    return out[None]


# speedup vs baseline: 7.1474x; 1.0038x over previous
import functools

import jax
import jax.numpy as jnp
from jax import lax
from jax.experimental import pallas as pl
from jax.experimental.pallas import tpu as pltpu
from jax.experimental.pallas import tpu_sc as plsc

IN_F = 2048
OUT_F = 2048
NCONN = 16

NUM_CORES = 2
NUM_SUBCORES = 16
NW = NUM_CORES * NUM_SUBCORES
ROWS_PER_W = OUT_F // NW
SUB = 2
NCHUNK = ROWS_PER_W // SUB


def _build_mt(conn_i32, weight, zeros_blk):
    mesh = plsc.VectorSubcoreMesh(core_axis_name="c", subcore_axis_name="s")

    @functools.partial(
        pl.kernel,
        mesh=mesh,
        out_type=jax.ShapeDtypeStruct((OUT_F, IN_F), jnp.float32),
        scratch_types=[
            pltpu.VMEM((SUB, IN_F), jnp.float32),
            pltpu.VMEM((SUB, IN_F), jnp.float32),
            pltpu.VMEM((ROWS_PER_W * NCONN,), jnp.int32),
            pltpu.VMEM((ROWS_PER_W, NCONN), jnp.float32),
            pltpu.SemaphoreType.DMA,
            pltpu.SemaphoreType.DMA,
            pltpu.SemaphoreType.DMA,
            pltpu.SemaphoreType.DMA,
        ],
        compiler_params=pltpu.CompilerParams(needs_layout_passes=False),
    )
    def k(conn_hbm, w_hbm, z_hbm, mt_hbm, blk0, blk1, idx_v, wv, s0, s1, s2,
          s3):
        wid = lax.axis_index("s") * NUM_CORES + lax.axis_index("c")
        base = wid * ROWS_PER_W
        blks = (blk0, blk1)
        sems = (s0, s1)
        z0 = pltpu.async_copy(z_hbm, blk0, s2)
        z1 = pltpu.async_copy(z_hbm, blk1, s3)
        pltpu.sync_copy(conn_hbm.at[pl.ds(base * NCONN, ROWS_PER_W * NCONN)],
                        idx_v)
        pltpu.sync_copy(w_hbm.at[pl.ds(base, ROWS_PER_W)], wv)
        z0.wait()
        z1.wait()
        zvec = jnp.zeros((NCONN,), jnp.float32)
        copies = [None, None]
        for c in range(NCHUNK):
            b = c % 2
            blk = blks[b]
            if copies[b] is not None:
                copies[b].wait()

                def _restore(r, _, blk=blk, c=c):
                    o = ((c - 2) * SUB + r) * NCONN
                    rvec = jnp.full((NCONN,), r, jnp.int32)
                    plsc.store_scatter(blk, [rvec, idx_v[pl.ds(o, NCONN)]],
                                       zvec)
                    return _

                lax.fori_loop(0, SUB, _restore, None, unroll=4)

            def _scatter(r, _, blk=blk, c=c):
                o = (c * SUB + r) * NCONN
                idx = idx_v[pl.ds(o, NCONN)]
                w = wv[c * SUB + r]
                rvec = jnp.full((NCONN,), r, jnp.int32)
                plsc.addupdate_scatter(blk, [rvec, idx], w)
                return _

            lax.fori_loop(0, SUB, _scatter, None, unroll=4)
            copies[b] = pltpu.async_copy(
                blk, mt_hbm.at[pl.ds(base + c * SUB, SUB)], sems[b])
        copies[0].wait()
        copies[1].wait()

    return k(conn_i32, weight, zeros_blk)


def _matmul(x2d, mt, bias):
    A_BLK = 256

    def body(x_ref, mt_ref, b_ref, o_ref):
        acc = lax.dot_general(
            x_ref[...], mt_ref[...],
            (((1,), (1,)), ((), ())),
            preferred_element_type=jnp.float32,
        )
        o_ref[...] = acc + b_ref[...]

    return pl.pallas_call(
        body,
        grid=(OUT_F // A_BLK,),
        in_specs=[
            pl.BlockSpec((2048, IN_F), lambda i: (0, 0)),
            pl.BlockSpec((A_BLK, IN_F), lambda i: (i, 0)),
            pl.BlockSpec((A_BLK,), lambda i: (i,)),
        ],
        out_specs=pl.BlockSpec((2048, A_BLK), lambda i: (0, i)),
        out_shape=jax.ShapeDtypeStruct((2048, OUT_F), jnp.float32),
    )(x2d, mt, bias)


def kernel(x, conn, weight, bias_param):
    conn_i = conn.astype(jnp.int32)
    zeros_blk = jnp.zeros((SUB, IN_F), jnp.float32)
    mt = _build_mt(conn_i, weight, zeros_blk)
    out = _matmul(x[0], mt, bias_param)
    return out[None]


# speedup vs baseline: 7.3460x; 1.0278x over previous
import functools

import jax
import jax.numpy as jnp
from jax import lax
from jax.experimental import pallas as pl
from jax.experimental.pallas import tpu as pltpu
from jax.experimental.pallas import tpu_sc as plsc

IN_F = 2048
OUT_F = 2048
NCONN = 16

NUM_CORES = 2
NUM_SUBCORES = 16
NW = NUM_CORES * NUM_SUBCORES
ROWS_PER_W = OUT_F // NW
SUB = 2
NCHUNK = ROWS_PER_W // SUB


def _build_mt(conn_i32, weight, zeros_blk):
    mesh = plsc.VectorSubcoreMesh(core_axis_name="c", subcore_axis_name="s")

    @functools.partial(
        pl.kernel,
        mesh=mesh,
        out_type=jax.ShapeDtypeStruct((OUT_F, IN_F), jnp.float32),
        scratch_types=[
            pltpu.VMEM((SUB, IN_F), jnp.float32),
            pltpu.VMEM((SUB, IN_F), jnp.float32),
            pltpu.VMEM((ROWS_PER_W * NCONN,), jnp.int32),
            pltpu.VMEM((ROWS_PER_W, NCONN), jnp.float32),
            pltpu.SemaphoreType.DMA,
            pltpu.SemaphoreType.DMA,
            pltpu.SemaphoreType.DMA,
            pltpu.SemaphoreType.DMA,
        ],
        compiler_params=pltpu.CompilerParams(needs_layout_passes=False),
    )
    def k(conn_hbm, w_hbm, z_hbm, mt_hbm, blk0, blk1, idx_v, wv, s0, s1, s2,
          s3):
        wid = lax.axis_index("s") * NUM_CORES + lax.axis_index("c")
        base = wid * ROWS_PER_W
        blks = (blk0, blk1)
        sems = (s0, s1)
        z0 = pltpu.async_copy(z_hbm, blk0, s2)
        z1 = pltpu.async_copy(z_hbm, blk1, s3)
        pltpu.sync_copy(conn_hbm.at[pl.ds(base * NCONN, ROWS_PER_W * NCONN)],
                        idx_v)
        pltpu.sync_copy(w_hbm.at[pl.ds(base, ROWS_PER_W)], wv)
        z0.wait()
        z1.wait()
        zvec = jnp.zeros((NCONN,), jnp.float32)

        def _chunk(g, _):
            for b in range(2):
                c = g * 2 + b
                blk = blks[b]

                @pl.when(g > 0)
                def _wait_and_restore(blk=blk, c=c, b=b):
                    pltpu.make_async_copy(
                        blk, mt_hbm.at[pl.ds(base + (c - 2) * SUB, SUB)],
                        sems[b]).wait()

                    def _restore(r, _):
                        o = ((c - 2) * SUB + r) * NCONN
                        rvec = jnp.full((NCONN,), r, jnp.int32)
                        plsc.store_scatter(blk,
                                           [rvec, idx_v[pl.ds(o, NCONN)]],
                                           zvec)
                        return _

                    lax.fori_loop(0, SUB, _restore, None, unroll=SUB)

                def _scatter(r, _, blk=blk, c=c):
                    o = (c * SUB + r) * NCONN
                    idx = idx_v[pl.ds(o, NCONN)]
                    w = wv[c * SUB + r]
                    rvec = jnp.full((NCONN,), r, jnp.int32)
                    plsc.addupdate_scatter(blk, [rvec, idx], w)
                    return _

                lax.fori_loop(0, SUB, _scatter, None, unroll=SUB)
                pltpu.async_copy(
                    blk, mt_hbm.at[pl.ds(base + c * SUB, SUB)], sems[b])
            return _

        lax.fori_loop(0, NCHUNK // 2, _chunk, None)
        for b in range(2):
            c_last = NCHUNK - 2 + b
            pltpu.make_async_copy(
                blks[b], mt_hbm.at[pl.ds(base + c_last * SUB, SUB)],
                sems[b]).wait()

    return k(conn_i32, weight, zeros_blk)


def _matmul(x2d, mt, bias):
    A_BLK = 256

    def body(x_ref, mt_ref, b_ref, o_ref):
        acc = lax.dot_general(
            x_ref[...], mt_ref[...],
            (((1,), (1,)), ((), ())),
            preferred_element_type=jnp.float32,
        )
        o_ref[...] = (acc + b_ref[...])[None]

    return pl.pallas_call(
        body,
        grid=(OUT_F // A_BLK,),
        in_specs=[
            pl.BlockSpec((2048, IN_F), lambda i: (0, 0)),
            pl.BlockSpec((A_BLK, IN_F), lambda i: (i, 0)),
            pl.BlockSpec((A_BLK,), lambda i: (i,)),
        ],
        out_specs=pl.BlockSpec((1, 2048, A_BLK), lambda i: (0, 0, i)),
        out_shape=jax.ShapeDtypeStruct((1, 2048, OUT_F), jnp.float32),
    )(x2d, mt, bias)


def kernel(x, conn, weight, bias_param):
    conn_i = conn.astype(jnp.int32)
    zeros_blk = jnp.zeros((SUB, IN_F), jnp.float32)
    mt = _build_mt(conn_i, weight, zeros_blk)
    return _matmul(x[0], mt, bias_param)
